# Initial kernel scaffold; baseline (speedup 1.0000x reference)
import jax
import jax.numpy as jnp
from jax.experimental import pallas as pl


def kernel(edge_index, user_emb, item_emb):
    raise NotImplementedError("write your pallas kernel here")


# speedup vs baseline: 5.2351x; 5.2351x over previous
import functools

import jax
import jax.numpy as jnp
from jax import lax
from jax.experimental import pallas as pl
from jax.experimental.pallas import tpu as pltpu
from jax.experimental.pallas import tpu_sc as plsc

N_USERS = 25000
N_NODES = 50000
D = 64
E = 800000
NUM_LAYERS = 4

NC = 2
NS = 16
NPAD = 50176
HALF = NPAD // 2
NSUB = NPAD // NS
NTILE = NPAD // 32
ACC_ROWS = HALF + 128
EPT = E // 32
EPS = E // 16
EB = 80
CH = 112

_mesh = functools.partial(
    plsc.VectorSubcoreMesh, core_axis_name="c", subcore_axis_name="s",
    num_cores=NC, num_subcores=NS)


def _rsqrt(x):
  i = lax.bitcast_convert_type(x, jnp.int32)
  i = jnp.int32(0x5F3759DF) - lax.shift_right_arithmetic(i, 1)
  y = lax.bitcast_convert_type(i, jnp.float32)
  for _ in range(4):
    y = y * (jnp.float32(1.5) - jnp.float32(0.5) * x * y * y)
  return y


def _zero_vec():
  return jnp.zeros((16,), jnp.float32)


def _k1_body(col, deg2, dstl, deg_sh, zb, ones, cb80, cb40, cbig, d0b, d1b):
  c = lax.axis_index("c")
  s = lax.axis_index("s")
  wid = s * NC + c
  ebase = pl.multiple_of(wid * EPT, 8)

  def _fill(i, _):
    zb[pl.ds(i * 16, 16)] = _zero_vec()
    return _
  lax.fori_loop(0, NSUB // 16, _fill, None)
  for j in range(EB // 16):
    ones[pl.ds(j * 16, 16)] = _zero_vec() + 1.0

  pltpu.sync_copy(zb, deg_sh.at[pl.ds(s * NSUB, NSUB)])
  plsc.subcore_barrier()

  def _deg_blk(b, _):
    e0 = pl.multiple_of(ebase + b * EB, 8)
    pltpu.sync_copy(col.at[pl.ds(e0, EB)], cb80)
    pltpu.sync_copy(ones, deg_sh.at[cb80], add=True)
    return _
  lax.fori_loop(0, EPT // EB, _deg_blk, None)
  e0 = pl.multiple_of(ebase + (EPT // EB) * EB, 8)
  pltpu.sync_copy(col.at[pl.ds(e0, 40)], cb40)
  pltpu.sync_copy(ones.at[pl.ds(0, 40)], deg_sh.at[cb40], add=True)

  iota = lax.iota(jnp.int32, 16)
  def _dst_chunk(ch, _):
    e0 = pl.multiple_of(ebase + ch * 1000, 8)
    pltpu.sync_copy(col.at[pl.ds(e0, 1000)], cbig.at[pl.ds(0, 1000)])
    def _dst_vec(i, _):
      col = cbig[pl.ds(i * 16, 16)]
      tr = HALF + ((e0 + i * 16) & 63) + iota
      d0b[pl.ds(i * 16, 16)] = jnp.where(col < HALF, col, tr)
      d1b[pl.ds(i * 16, 16)] = jnp.where(col >= HALF, col - HALF, tr)
      return _
    lax.fori_loop(0, 63, _dst_vec, None)
    pltpu.sync_copy(d0b.at[pl.ds(0, 1000)], dstl.at[pl.ds(e0, 1000)])
    pltpu.sync_copy(d1b.at[pl.ds(0, 1000)], dstl.at[pl.ds(E + e0, 1000)])
    return _
  lax.fori_loop(0, EPT // 1000, _dst_chunk, None)

  plsc.subcore_barrier()
  pltpu.sync_copy(deg_sh.at[pl.ds(s * NSUB, NSUB)], zb)
  pltpu.sync_copy(zb, deg2.at[pl.ds(c * NPAD + s * NSUB, NSUB)])


_k1 = pl.kernel(
    _k1_body,
    out_type=(jax.ShapeDtypeStruct((NC * NPAD,), jnp.float32),
              jax.ShapeDtypeStruct((2 * E,), jnp.int32)),
    mesh=_mesh(),
    compiler_params=pltpu.CompilerParams(use_tc_tiling_on_sc=False),
    scratch_types=[
        pltpu.VMEM_SHARED((NPAD,), jnp.float32),
        pltpu.VMEM((NSUB,), jnp.float32),
        pltpu.VMEM((EB,), jnp.float32),
        pltpu.VMEM((EB,), jnp.int32),
        pltpu.VMEM((40,), jnp.int32),
        pltpu.VMEM((1008,), jnp.int32),
        pltpu.VMEM((1008,), jnp.int32),
        pltpu.VMEM((1008,), jnp.int32),
    ],
)


def _k2_body(deg2, x0, s_out, g0, f0, db0, db1, sb, xb, fb):
  c = lax.axis_index("c")
  s = lax.axis_index("s")
  wid = s * NC + c
  nbase = pl.multiple_of(wid * NTILE, 8)

  pltpu.sync_copy(deg2.at[pl.ds(nbase, NTILE)], db0)
  pltpu.sync_copy(deg2.at[pl.ds(NPAD + nbase, NTILE)], db1)

  def _svec(i, _):
    d = db0[pl.ds(i * 16, 16)] + db1[pl.ds(i * 16, 16)]
    y = _rsqrt(jnp.maximum(d, 1.0))
    sb[pl.ds(i * 16, 16)] = jnp.where(d > 0, y, 0.0)
    return _
  lax.fori_loop(0, NTILE // 16, _svec, None)
  pltpu.sync_copy(sb, s_out.at[pl.ds(nbase, NTILE)])

  def _chunk(k, _):
    off = pl.multiple_of(nbase + k * CH, 8)
    pltpu.sync_copy(x0.at[pl.ds(off, CH)], xb)
    def _rowgrp(rg, _):
      sv16 = sb[pl.ds(k * CH + rg * 16, 16)]
      for i in range(16):
        r = rg * 16 + i
        sv = sv16[i]
        for j in range(D // 16):
          xv = xb[r, pl.ds(j * 16, 16)]
          fb[r, pl.ds(j * 16, 16)] = xv * 0.2
          xb[r, pl.ds(j * 16, 16)] = xv * sv
      return _
    lax.fori_loop(0, CH // 16, _rowgrp, None)
    pltpu.sync_copy(fb, f0.at[pl.ds(off, CH)])
    pltpu.sync_copy(xb, g0.at[pl.ds(off, CH)])
    return _
  lax.fori_loop(0, NTILE // CH, _chunk, None)


_k2 = pl.kernel(
    _k2_body,
    out_type=(jax.ShapeDtypeStruct((NPAD,), jnp.float32),
              jax.ShapeDtypeStruct((NPAD, D), jnp.float32),
              jax.ShapeDtypeStruct((NPAD, D), jnp.float32)),
    mesh=_mesh(),
    compiler_params=pltpu.CompilerParams(use_tc_tiling_on_sc=False),
    scratch_types=[
        pltpu.VMEM((NTILE,), jnp.float32),
        pltpu.VMEM((NTILE,), jnp.float32),
        pltpu.VMEM((NTILE,), jnp.float32),
        pltpu.VMEM((CH, D), jnp.float32),
        pltpu.VMEM((CH, D), jnp.float32),
    ],
)


def _k3_body(row, dstl, g_in, s_arr, f_in, g_out, f_out,
             acc, zb, rowb, dstb, gb, fb, svb):
  c = lax.axis_index("c")
  s = lax.axis_index("s")

  def _fillz(r, _):
    for j in range(D // 16):
      zb[r, pl.ds(j * 16, 16)] = _zero_vec()
    return _
  lax.fori_loop(0, CH, _fillz, None)
  zbase = s * (ACC_ROWS // NS)
  for k in range(14):
    pltpu.sync_copy(zb, acc.at[pl.ds(zbase + k * CH, CH)])
  pltpu.sync_copy(zb.at[pl.ds(0, 8)], acc.at[pl.ds(zbase + 14 * CH, 8)])
  plsc.subcore_barrier()

  ebase = pl.multiple_of(s * EPS, 8)
  def _edge_blk(b, _):
    e0 = pl.multiple_of(ebase + b * EB, 8)
    pltpu.sync_copy(row.at[pl.ds(e0, EB)], rowb)
    pltpu.sync_copy(dstl.at[pl.ds(c * E + e0, EB)], dstb)
    pltpu.sync_copy(g_in.at[rowb], gb)
    pltpu.sync_copy(gb, acc.at[dstb], add=True)
    return _
  lax.fori_loop(0, EPS // EB, _edge_blk, None)
  plsc.subcore_barrier()

  lbase = s * NTILE
  gbase = pl.multiple_of(c * HALF + lbase, 8)
  def _flush_chunk(k, _):
    loff = lbase + k * CH
    goff = pl.multiple_of(gbase + k * CH, 8)
    pltpu.sync_copy(acc.at[pl.ds(loff, CH)], zb)
    pltpu.sync_copy(f_in.at[pl.ds(goff, CH)], fb)
    pltpu.sync_copy(s_arr.at[pl.ds(goff, CH)], svb)
    def _rowgrp(rg, _):
      sv16 = svb[pl.ds(rg * 16, 16)]
      for i in range(16):
        r = rg * 16 + i
        sv = sv16[i]
        a = sv * 0.2
        s2 = sv * sv
        for j in range(D // 16):
          zv = zb[r, pl.ds(j * 16, 16)]
          fb[r, pl.ds(j * 16, 16)] = fb[r, pl.ds(j * 16, 16)] + a * zv
          zb[r, pl.ds(j * 16, 16)] = s2 * zv
      return _
    lax.fori_loop(0, CH // 16, _rowgrp, None)
    pltpu.sync_copy(fb, f_out.at[pl.ds(goff, CH)])
    pltpu.sync_copy(zb, g_out.at[pl.ds(goff, CH)])
    return _
  lax.fori_loop(0, NTILE // CH, _flush_chunk, None)


_k3 = pl.kernel(
    _k3_body,
    out_type=(jax.ShapeDtypeStruct((NPAD, D), jnp.float32),
              jax.ShapeDtypeStruct((NPAD, D), jnp.float32)),
    mesh=_mesh(),
    compiler_params=pltpu.CompilerParams(use_tc_tiling_on_sc=False),
    scratch_types=[
        pltpu.VMEM_SHARED((ACC_ROWS, D), jnp.float32),
        pltpu.VMEM((CH, D), jnp.float32),
        pltpu.VMEM((EB,), jnp.int32),
        pltpu.VMEM((EB,), jnp.int32),
        pltpu.VMEM((EB, D), jnp.float32),
        pltpu.VMEM((CH, D), jnp.float32),
        pltpu.VMEM((CH,), jnp.float32),
    ],
)


def kernel(edge_index, user_emb, item_emb):
  x0 = jnp.concatenate([user_emb, item_emb], axis=0)
  x0 = jnp.pad(x0, ((0, NPAD - N_NODES), (0, 0)))
  erow = edge_index[0]
  ecol = edge_index[1]
  deg2, dstl = _k1(ecol)
  s_arr, g, f = _k2(deg2, x0)
  for _ in range(NUM_LAYERS):
    g, f = _k3(erow, dstl, g, s_arr, f)
  out = f[:N_NODES]
  return out[:N_USERS], out[N_USERS:]
